# baseline (speedup 1.0000x reference)
import functools

import jax
import jax.numpy as jnp
from jax import lax
from jax.experimental import pallas as pl
from jax.experimental.pallas import tpu as pltpu
from jax.experimental.pallas import tpu_sc as plsc

_NUM_CORES = 2
_NUM_SUBCORES = 16
_NUM_WORKERS = _NUM_CORES * _NUM_SUBCORES
_CHUNK = 400
_D_OUT = 128


@functools.partial(jax.jit, static_argnames=("n_chunks", "d_word", "d_pos"))
def _embed(word_i, pos1_i, pos2_i, word_table, pos1_table, pos2_table,
           n_chunks, d_word, d_pos):
    n_total = _NUM_WORKERS * n_chunks * _CHUNK
    mesh = plsc.VectorSubcoreMesh(core_axis_name="c", subcore_axis_name="s")

    @functools.partial(
        pl.kernel,
        mesh=mesh,
        compiler_params=pltpu.CompilerParams(use_tc_tiling_on_sc=False),
        out_type=jax.ShapeDtypeStruct((n_total, _D_OUT), jnp.float32),
        scratch_types=[
            pltpu.VMEM((n_chunks, _CHUNK), jnp.int32),
            pltpu.VMEM((n_chunks, _CHUNK), jnp.int32),
            pltpu.VMEM((n_chunks, _CHUNK), jnp.int32),
            pltpu.VMEM((_CHUNK, _D_OUT), jnp.float32),
            pltpu.VMEM((_CHUNK, 16), jnp.float32),
            pltpu.VMEM((_CHUNK, 16), jnp.float32),
            pltpu.SemaphoreType.DMA,
            pltpu.SemaphoreType.DMA,
            pltpu.SemaphoreType.DMA,
        ],
    )
    def emb_kernel(w_hbm, p1_hbm, p2_hbm, wt_hbm, p1t_hbm, p2t_hbm, out_hbm,
                   widx, p1idx, p2idx, wbuf, p1buf, p2buf, semw, sem1, sem2):
        wid = lax.axis_index("s") * _NUM_CORES + lax.axis_index("c")
        pltpu.sync_copy(w_hbm.at[wid], widx)
        pltpu.sync_copy(p1_hbm.at[wid], p1idx)
        pltpu.sync_copy(p2_hbm.at[wid], p2idx)
        base0 = wid * (n_chunks * _CHUNK)

        def body(j, carry):
            cw = pltpu.async_copy(wt_hbm.at[widx.at[j]], wbuf, semw)
            c1 = pltpu.async_copy(p1t_hbm.at[p1idx.at[j]], p1buf, sem1)
            c2 = pltpu.async_copy(p2t_hbm.at[p2idx.at[j]], p2buf, sem2)
            cw.wait()
            c1.wait()
            c2.wait()
            base = base0 + j * _CHUNK
            pltpu.sync_copy(wbuf.at[:, pl.ds(0, d_word)],
                            out_hbm.at[pl.ds(base, _CHUNK), pl.ds(0, d_word)])
            pltpu.sync_copy(p1buf, out_hbm.at[pl.ds(base, _CHUNK),
                                              pl.ds(d_word, d_pos)])
            pltpu.sync_copy(p2buf, out_hbm.at[pl.ds(base, _CHUNK),
                                              pl.ds(d_word + d_pos, d_pos)])
            return carry

        lax.fori_loop(0, n_chunks, body, 0)

    return emb_kernel(word_i, pos1_i, pos2_i,
                      word_table, pos1_table, pos2_table)


def kernel(word, pos1, pos2, word_table, pos1_table, pos2_table):
    b, l = word.shape
    d_word = word_table.shape[1]
    d_pos = pos1_table.shape[1]
    n = b * l
    assert n % (_NUM_WORKERS * _CHUNK) == 0
    n_chunks = n // (_NUM_WORKERS * _CHUNK)

    shape = (_NUM_WORKERS, n_chunks, _CHUNK)
    word_i = word.reshape(shape).astype(jnp.int32)
    pos1_i = pos1.reshape(shape).astype(jnp.int32)
    pos2_i = pos2.reshape(shape).astype(jnp.int32)
    vocab = word_table.shape[0]
    word_table128 = jnp.concatenate(
        [word_table,
         jnp.zeros((vocab, _D_OUT - d_word), jnp.float32)], axis=1)

    out = _embed(word_i, pos1_i, pos2_i,
                 word_table128, pos1_table, pos2_table,
                 n_chunks, d_word, d_pos)
    return out[:, :d_word + 2 * d_pos].reshape(b, l, d_word + 2 * d_pos)
